# baseline (speedup 1.0000x reference)
import functools

import jax
import jax.numpy as jnp
from jax import lax
from jax.experimental import pallas as pl
from jax.experimental.pallas import tpu as pltpu
from jax.experimental.pallas import tpu_sc as plsc

N = 10000
E = 320000
D = 128
H = 64

_NC = 2
_NS = 16
_NW = _NC * _NS
_CPB = 128
_NCH = E // _CPB
_WCH = 80
_NR = 10016
_RPS = _NR // _NS
_TRS = N // _NS
_DW = 16


def _make_agg(with_deg):
    W = H
    mesh = plsc.VectorSubcoreMesh(core_axis_name="c", subcore_axis_name="s")
    out_type = [jax.ShapeDtypeStruct((_NC, _NR, W), jnp.float32)]
    scratch = [
        pltpu.VMEM((_WCH, _CPB), jnp.int32),
        pltpu.VMEM((_WCH, _CPB), jnp.int32),
        pltpu.VMEM((2 * _CPB, W), jnp.float32),
        pltpu.VMEM_SHARED((N, W), jnp.float32),
        pltpu.VMEM_SHARED((_NR, W), jnp.float32),
        pltpu.SemaphoreType.DMA,
        pltpu.SemaphoreType.DMA,
    ]
    if with_deg:
        out_type.append(jax.ShapeDtypeStruct((_NC, _NR, _DW), jnp.float32))
        scratch.append(pltpu.VMEM((_CPB, _DW), jnp.float32))
        scratch.append(pltpu.VMEM_SHARED((_NR, _DW), jnp.float32))

    @functools.partial(
        pl.kernel,
        mesh=mesh,
        compiler_params=pltpu.CompilerParams(use_tc_tiling_on_sc=False),
        out_type=tuple(out_type),
        scratch_types=scratch,
    )
    def agg(y_hbm, ei_hbm, p_hbm, *rest):
        if with_deg:
            pd_hbm, srcv, dstv, rows, ytab, acc, sem_g, sem_s, ones, dacc = rest
        else:
            srcv, dstv, rows, ytab, acc, sem_g, sem_s = rest
        c = lax.axis_index("c")
        s = lax.axis_index("s")
        wid = s * _NC + c

        pltpu.sync_copy(y_hbm.at[pl.ds(s * _TRS, _TRS)],
                        ytab.at[pl.ds(s * _TRS, _TRS)])

        base = wid * _WCH
        nch = jnp.minimum(_WCH, _NCH - base)
        base_l = jnp.minimum(base, _NCH - _WCH)
        off = base - base_l
        pltpu.sync_copy(ei_hbm.at[0, pl.ds(base_l, _WCH)], srcv)
        pltpu.sync_copy(ei_hbm.at[1, pl.ds(base_l, _WCH)], dstv)

        zeros16 = jnp.zeros((16,), jnp.float32)

        def zbody(r, carry):
            for k2 in range(W // 16):
                rows[r, pl.ds(k2 * 16, 16)] = zeros16
            return carry

        lax.fori_loop(0, _CPB, zbody, 0)
        full = _RPS // _CPB
        rem = _RPS - full * _CPB
        for t in range(full):
            pltpu.sync_copy(rows.at[pl.ds(0, _CPB)],
                            acc.at[pl.ds(s * _RPS + t * _CPB, _CPB)])
        if rem:
            pltpu.sync_copy(rows.at[pl.ds(0, rem)],
                            acc.at[pl.ds(s * _RPS + full * _CPB, rem)])

        if with_deg:
            def dbody(r, carry):
                ones[r, pl.ds(0, 16)] = zeros16
                return carry

            lax.fori_loop(0, _CPB, dbody, 0)
            for t in range(full):
                pltpu.sync_copy(ones.at[pl.ds(0, _CPB)],
                                dacc.at[pl.ds(s * _RPS + t * _CPB, _CPB)])
            if rem:
                pltpu.sync_copy(ones.at[pl.ds(0, rem)],
                                dacc.at[pl.ds(s * _RPS + full * _CPB, rem)])

            ones16 = jnp.full((16,), 1.0, jnp.float32)

            def obody(r, carry):
                ones[r, pl.ds(0, 16)] = ones16
                return carry

            lax.fori_loop(0, _CPB, obody, 0)

        plsc.subcore_barrier()

        def buf(g):
            return rows.at[pl.ds((g % 2) * _CPB, _CPB)]

        def fire_gather(g):
            pltpu.async_copy(ytab.at[srcv.at[g + off]], buf(g), sem_g)

        def drain_gather(g):
            pltpu.make_async_copy(ytab.at[srcv.at[0]], buf(g), sem_g).wait()

        def fire_scatter(g):
            pltpu.async_copy(buf(g), acc.at[dstv.at[g + off]], sem_s,
                             add=True)
            if with_deg:
                pltpu.async_copy(ones, dacc.at[dstv.at[g + off]], sem_s,
                                 add=True)

        def drain_scatter(g):
            pltpu.make_async_copy(buf(g), acc.at[dstv.at[0]], sem_s).wait()
            if with_deg:
                pltpu.make_async_copy(ones, dacc.at[dstv.at[0]],
                                      sem_s).wait()

        fire_gather(0)

        def body(g, carry):
            drain_gather(g)

            @pl.when(g >= 1)
            def _():
                drain_scatter(g - 1)

            @pl.when(g + 1 < nch)
            def _():
                fire_gather(g + 1)

            fire_scatter(g)
            return carry

        lax.fori_loop(0, nch, body, 0)
        drain_scatter(nch - 1)

        plsc.subcore_barrier()
        pltpu.sync_copy(acc.at[pl.ds(s * _RPS, _RPS)],
                        p_hbm.at[c, pl.ds(s * _RPS, _RPS)])
        if with_deg:
            pltpu.sync_copy(dacc.at[pl.ds(s * _RPS, _RPS)],
                            pd_hbm.at[c, pl.ds(s * _RPS, _RPS)])

    return agg


_agg_l1 = _make_agg(True)
_agg_l2 = _make_agg(False)


_BLK = 2000


def _body_a(x_ref, wl_ref, y_ref):
    y_ref[...] = jnp.dot(x_ref[...], wl_ref[...],
                         preferred_element_type=jnp.float32)


def _body_c(p_ref, pd_ref, x_ref, wr1_ref, b1_ref, wl_ref, wr_ref, b_ref,
            y2_ref, z2_ref, inv_ref):
    ps = p_ref[0] + p_ref[1]
    degs = pd_ref[0] + pd_ref[1]
    deg = jnp.sum(degs, axis=1, keepdims=True) * (1.0 / _DW)
    inv = 1.0 / jnp.maximum(deg, 1.0)
    z1 = jnp.dot(x_ref[...], wr1_ref[...],
                 preferred_element_type=jnp.float32) + b1_ref[...]
    h = jnp.maximum(ps * inv + z1, 0.0)
    y2_ref[...] = jnp.dot(h, wl_ref[...], preferred_element_type=jnp.float32)
    z2_ref[...] = jnp.dot(h, wr_ref[...],
                          preferred_element_type=jnp.float32) + b_ref[...]
    inv_ref[...] = inv


def _body_d(p_ref, z_ref, inv_ref, wc_ref, bc_ref, o_ref):
    ps = p_ref[0] + p_ref[1]
    h = jnp.maximum(ps * inv_ref[...] + z_ref[...], 0.0)
    o_ref[...] = jnp.dot(h, wc_ref[...],
                         preferred_element_type=jnp.float32) + bc_ref[...]


def kernel(x, edge_index, Wl1, bl1, Wr1, Wl2, bl2, Wr2, Wc, bc):
    f32 = jnp.float32
    ei = edge_index.reshape(2, _NCH, _CPB)

    b1 = bl1.reshape(1, H)
    b2 = bl2.reshape(1, H)
    C = Wc.shape[1]
    bcr = bc.reshape(1, C)
    grid = (N // _BLK,)

    def _full(shape):
        return pl.BlockSpec(shape, lambda i: (0,) * len(shape))

    def _rows(shape):
        return pl.BlockSpec(shape, lambda i: (i,) + (0,) * (len(shape) - 1))

    def _prows(shape):
        return pl.BlockSpec(shape, lambda i: (0, i) + (0,) * (len(shape) - 2))

    y1 = pl.pallas_call(
        _body_a,
        grid=grid,
        in_specs=[_rows((_BLK, D)), _full((D, H))],
        out_specs=_rows((_BLK, H)),
        out_shape=jax.ShapeDtypeStruct((N, H), f32),
    )(x, Wl1)

    p1, pdeg = _agg_l1(y1, ei)

    y2, z2, inv = pl.pallas_call(
        _body_c,
        grid=grid,
        in_specs=[_prows((_NC, _BLK, H)), _prows((_NC, _BLK, _DW)),
                  _rows((_BLK, D)), _full((D, H)), _full((1, H)),
                  _full((H, H)), _full((H, H)), _full((1, H))],
        out_specs=(_rows((_BLK, H)), _rows((_BLK, H)), _rows((_BLK, 1))),
        out_shape=(jax.ShapeDtypeStruct((N, H), f32),
                   jax.ShapeDtypeStruct((N, H), f32),
                   jax.ShapeDtypeStruct((N, 1), f32)),
    )(p1, pdeg, x, Wr1, b1, Wl2, Wr2, b2)

    p2, = _agg_l2(y2, ei)

    out = pl.pallas_call(
        _body_d,
        grid=grid,
        in_specs=[_prows((_NC, _BLK, H)), _rows((_BLK, H)), _rows((_BLK, 1)),
                  _full((H, C)), _full((1, C))],
        out_specs=_rows((_BLK, C)),
        out_shape=jax.ShapeDtypeStruct((N, C), f32),
    )(p2, z2, inv, Wc, bcr)

    return out


# speedup vs baseline: 1.0387x; 1.0387x over previous
import functools

import jax
import jax.numpy as jnp
from jax import lax
from jax.experimental import pallas as pl
from jax.experimental.pallas import tpu as pltpu
from jax.experimental.pallas import tpu_sc as plsc

N = 10000
E = 320000
D = 128
H = 64

_NC = 2
_NS = 16
_NW = _NC * _NS
_CPB = 128
_NCH = E // _CPB
_WCH = 80
_NR = 10016
_RPS = _NR // _NS
_TRS = N // _NS
_DW = 16


def _make_agg(with_deg):
    W = H
    mesh = plsc.VectorSubcoreMesh(core_axis_name="c", subcore_axis_name="s")
    out_type = [jax.ShapeDtypeStruct((_NC, _NR, W), jnp.float32)]
    scratch = [
        pltpu.VMEM((_WCH, _CPB), jnp.int32),
        pltpu.VMEM((_WCH, _CPB), jnp.int32),
        pltpu.VMEM((2 * _CPB, W), jnp.float32),
        pltpu.VMEM_SHARED((N, W), jnp.float32),
        pltpu.VMEM_SHARED((_NR, W), jnp.float32),
        pltpu.SemaphoreType.DMA,
        pltpu.SemaphoreType.DMA,
    ]
    if with_deg:
        out_type.append(jax.ShapeDtypeStruct((_NC, _NR, _DW), jnp.float32))
        scratch.append(pltpu.VMEM((_CPB, _DW), jnp.float32))
        scratch.append(pltpu.VMEM_SHARED((_NR, _DW), jnp.float32))

    @functools.partial(
        pl.kernel,
        mesh=mesh,
        compiler_params=pltpu.CompilerParams(use_tc_tiling_on_sc=False),
        out_type=tuple(out_type),
        scratch_types=scratch,
    )
    def agg(y_hbm, ei_hbm, p_hbm, *rest):
        if with_deg:
            pd_hbm, srcv, dstv, rows, ytab, acc, sem_g, sem_s, ones, dacc = rest
        else:
            srcv, dstv, rows, ytab, acc, sem_g, sem_s = rest
        c = lax.axis_index("c")
        s = lax.axis_index("s")
        wid = s * _NC + c

        pltpu.async_copy(y_hbm.at[pl.ds(s * _TRS, _TRS)],
                         ytab.at[pl.ds(s * _TRS, _TRS)], sem_g)

        base = wid * _WCH
        nch = jnp.minimum(_WCH, _NCH - base)
        base_l = jnp.minimum(base, _NCH - _WCH)
        off = base - base_l
        pltpu.async_copy(ei_hbm.at[0, pl.ds(base_l, _WCH)], srcv, sem_s)
        pltpu.async_copy(ei_hbm.at[1, pl.ds(base_l, _WCH)], dstv, sem_s)

        zeros16 = jnp.zeros((16,), jnp.float32)

        def zbody(r, carry):
            for k2 in range(W // 16):
                rows[r, pl.ds(k2 * 16, 16)] = zeros16
            return carry

        lax.fori_loop(0, _CPB, zbody, 0)
        full = _RPS // _CPB
        rem = _RPS - full * _CPB
        for t in range(full):
            pltpu.sync_copy(rows.at[pl.ds(0, _CPB)],
                            acc.at[pl.ds(s * _RPS + t * _CPB, _CPB)])
        if rem:
            pltpu.sync_copy(rows.at[pl.ds(0, rem)],
                            acc.at[pl.ds(s * _RPS + full * _CPB, rem)])

        if with_deg:
            def dbody(r, carry):
                ones[r, pl.ds(0, 16)] = zeros16
                return carry

            lax.fori_loop(0, _CPB, dbody, 0)
            for t in range(full):
                pltpu.sync_copy(ones.at[pl.ds(0, _CPB)],
                                dacc.at[pl.ds(s * _RPS + t * _CPB, _CPB)])
            if rem:
                pltpu.sync_copy(ones.at[pl.ds(0, rem)],
                                dacc.at[pl.ds(s * _RPS + full * _CPB, rem)])

            ones16 = jnp.full((16,), 1.0, jnp.float32)

            def obody(r, carry):
                ones[r, pl.ds(0, 16)] = ones16
                return carry

            lax.fori_loop(0, _CPB, obody, 0)

        pltpu.make_async_copy(y_hbm.at[pl.ds(s * _TRS, _TRS)],
                              ytab.at[pl.ds(s * _TRS, _TRS)], sem_g).wait()
        pltpu.make_async_copy(ei_hbm.at[0, pl.ds(base_l, _WCH)], srcv,
                              sem_s).wait()
        pltpu.make_async_copy(ei_hbm.at[1, pl.ds(base_l, _WCH)], dstv,
                              sem_s).wait()
        plsc.subcore_barrier()

        def buf(g):
            return rows.at[pl.ds((g % 2) * _CPB, _CPB)]

        def fire_gather(g):
            pltpu.async_copy(ytab.at[srcv.at[g + off]], buf(g), sem_g)

        def drain_gather(g):
            pltpu.make_async_copy(ytab.at[srcv.at[0]], buf(g), sem_g).wait()

        def fire_scatter(g):
            pltpu.async_copy(buf(g), acc.at[dstv.at[g + off]], sem_s,
                             add=True)
            if with_deg:
                pltpu.async_copy(ones, dacc.at[dstv.at[g + off]], sem_s,
                                 add=True)

        def drain_scatter(g):
            pltpu.make_async_copy(buf(g), acc.at[dstv.at[0]], sem_s).wait()
            if with_deg:
                pltpu.make_async_copy(ones, dacc.at[dstv.at[0]],
                                      sem_s).wait()

        fire_gather(0)

        def body(g, carry):
            drain_gather(g)

            @pl.when(g >= 1)
            def _():
                drain_scatter(g - 1)

            @pl.when(g + 1 < nch)
            def _():
                fire_gather(g + 1)

            fire_scatter(g)
            return carry

        lax.fori_loop(0, nch, body, 0)
        drain_scatter(nch - 1)

        plsc.subcore_barrier()
        pltpu.async_copy(acc.at[pl.ds(s * _RPS, _RPS)],
                         p_hbm.at[c, pl.ds(s * _RPS, _RPS)], sem_g)
        if with_deg:
            pltpu.async_copy(dacc.at[pl.ds(s * _RPS, _RPS)],
                             pd_hbm.at[c, pl.ds(s * _RPS, _RPS)], sem_s)
        pltpu.make_async_copy(acc.at[pl.ds(s * _RPS, _RPS)],
                              p_hbm.at[c, pl.ds(s * _RPS, _RPS)],
                              sem_g).wait()
        if with_deg:
            pltpu.make_async_copy(dacc.at[pl.ds(s * _RPS, _RPS)],
                                  pd_hbm.at[c, pl.ds(s * _RPS, _RPS)],
                                  sem_s).wait()

    return agg


_agg_l1 = _make_agg(True)
_agg_l2 = _make_agg(False)


_BLK = 2000


def _body_a(x_ref, wl_ref, y_ref):
    y_ref[...] = jnp.dot(x_ref[...], wl_ref[...],
                         preferred_element_type=jnp.float32)


def _body_c(p_ref, pd_ref, x_ref, wr1_ref, b1_ref, wl_ref, wr_ref, b_ref,
            y2_ref, z2_ref, inv_ref):
    ps = p_ref[0] + p_ref[1]
    degs = pd_ref[0] + pd_ref[1]
    deg = jnp.sum(degs, axis=1, keepdims=True) * (1.0 / _DW)
    inv = 1.0 / jnp.maximum(deg, 1.0)
    z1 = jnp.dot(x_ref[...], wr1_ref[...],
                 preferred_element_type=jnp.float32) + b1_ref[...]
    h = jnp.maximum(ps * inv + z1, 0.0)
    y2_ref[...] = jnp.dot(h, wl_ref[...], preferred_element_type=jnp.float32)
    z2_ref[...] = jnp.dot(h, wr_ref[...],
                          preferred_element_type=jnp.float32) + b_ref[...]
    inv_ref[...] = inv


def _body_d(p_ref, z_ref, inv_ref, wc_ref, bc_ref, o_ref):
    ps = p_ref[0] + p_ref[1]
    h = jnp.maximum(ps * inv_ref[...] + z_ref[...], 0.0)
    o_ref[...] = jnp.dot(h, wc_ref[...],
                         preferred_element_type=jnp.float32) + bc_ref[...]


def kernel(x, edge_index, Wl1, bl1, Wr1, Wl2, bl2, Wr2, Wc, bc):
    f32 = jnp.float32
    ei = edge_index.reshape(2, _NCH, _CPB)

    b1 = bl1.reshape(1, H)
    b2 = bl2.reshape(1, H)
    C = Wc.shape[1]
    bcr = bc.reshape(1, C)
    grid = (N // _BLK,)

    def _full(shape):
        return pl.BlockSpec(shape, lambda i: (0,) * len(shape))

    def _rows(shape):
        return pl.BlockSpec(shape, lambda i: (i,) + (0,) * (len(shape) - 1))

    def _prows(shape):
        return pl.BlockSpec(shape, lambda i: (0, i) + (0,) * (len(shape) - 2))

    y1 = pl.pallas_call(
        _body_a,
        grid=grid,
        in_specs=[_rows((_BLK, D)), _full((D, H))],
        out_specs=_rows((_BLK, H)),
        out_shape=jax.ShapeDtypeStruct((N, H), f32),
    )(x, Wl1)

    p1, pdeg = _agg_l1(y1, ei)

    y2, z2, inv = pl.pallas_call(
        _body_c,
        grid=grid,
        in_specs=[_prows((_NC, _BLK, H)), _prows((_NC, _BLK, _DW)),
                  _rows((_BLK, D)), _full((D, H)), _full((1, H)),
                  _full((H, H)), _full((H, H)), _full((1, H))],
        out_specs=(_rows((_BLK, H)), _rows((_BLK, H)), _rows((_BLK, 1))),
        out_shape=(jax.ShapeDtypeStruct((N, H), f32),
                   jax.ShapeDtypeStruct((N, H), f32),
                   jax.ShapeDtypeStruct((N, 1), f32)),
    )(p1, pdeg, x, Wr1, b1, Wl2, Wr2, b2)

    p2, = _agg_l2(y2, ei)

    out = pl.pallas_call(
        _body_d,
        grid=grid,
        in_specs=[_prows((_NC, _BLK, H)), _rows((_BLK, H)), _rows((_BLK, 1)),
                  _full((H, C)), _full((1, C))],
        out_specs=_rows((_BLK, C)),
        out_shape=jax.ShapeDtypeStruct((N, C), f32),
    )(p2, z2, inv, Wc, bcr)

    return out
